# Initial kernel scaffold; baseline (speedup 1.0000x reference)
import jax
import jax.numpy as jnp
from jax.experimental import pallas as pl


def kernel(cnn_encoder_output, original_input, xy, g_w, g_b, g_a, q_w, q_b, q_a, conv_w, conv_b, gnn_iterations, k, use_half_precision):
    raise NotImplementedError("write your pallas kernel here")


# speedup vs baseline: 3.6323x; 3.6323x over previous
import functools

import jax
import jax.numpy as jnp
from jax import lax
from jax.experimental import pallas as pl
from jax.experimental.pallas import tpu as pltpu
from jax.experimental.pallas import tpu_sc as plsc

HW = 784
KNN = 16
N_IMG = 4
C_FEAT = 128
B_ROWS = N_IMG * HW
NW = 32
RPW = B_ROWS // NW
LANES = 16
CCHUNKS = C_FEAT // LANES


def _median_body(x_ref, o_ref):
    v = x_ref[:]
    r = v.shape[0]
    col = lax.broadcasted_iota(jnp.int32, (r, 64), 1)
    med = jnp.zeros((r, 1), jnp.float32)
    for j in range(64):
        vj = v[:, j:j + 1]
        lt = (v < vj).astype(jnp.float32)
        eq_before = ((v == vj) & (col < j)).astype(jnp.float32)
        cnt = jnp.sum(lt + eq_before, axis=1, keepdims=True)
        med = med + jnp.where(cnt == 31.0, vj, 0.0)
    o_ref[:] = med


def _median_pool(blocks):
    rows = blocks.shape[0]
    grid = 12
    blk = rows // grid
    return pl.pallas_call(
        _median_body,
        grid=(grid,),
        in_specs=[pl.BlockSpec((blk, 64), lambda i: (i, 0))],
        out_specs=pl.BlockSpec((blk, 1), lambda i: (i, 0)),
        out_shape=jax.ShapeDtypeStruct((rows, 1), jnp.float32),
    )(blocks)


def _knn_body(p_ref, pt_ref, o_ref):
    p = p_ref[0]
    pt = pt_ref[0]
    r = p[:, 0:1] * pt[0:1, :]
    r = r + p[:, 1:2] * pt[1:2, :]
    r = r + p[:, 2:3] * pt[2:3, :]
    dcol = p[:, 0:1] * p[:, 0:1]
    dcol = dcol + p[:, 1:2] * p[:, 1:2]
    dcol = dcol + p[:, 2:3] * p[:, 2:3]
    drow = pt[0:1, :] * pt[0:1, :]
    drow = drow + pt[1:2, :] * pt[1:2, :]
    drow = drow + pt[2:3, :] * pt[2:3, :]
    d2 = (dcol + drow) - 2.0 * r
    d = jnp.sqrt(jnp.maximum(d2, 0.0))
    col = lax.broadcasted_iota(jnp.int32, (HW, HW), 1)
    big = jnp.float32(jnp.inf)
    idxs = []
    for _ in range(KNN):
        vmin = jnp.min(d, axis=1, keepdims=True)
        cand = jnp.where(d == vmin, col, HW)
        idx = jnp.min(cand, axis=1, keepdims=True)
        idxs.append(idx)
        d = jnp.where(col == idx, big, d)
    o_ref[0] = jnp.concatenate(idxs, axis=1)


def _knn(proj, projt):
    return pl.pallas_call(
        _knn_body,
        grid=(N_IMG,),
        in_specs=[
            pl.BlockSpec((1, HW, 3), lambda n: (n, 0, 0)),
            pl.BlockSpec((1, 3, HW), lambda n: (n, 0, 0)),
        ],
        out_specs=pl.BlockSpec((1, HW, KNN), lambda n: (n, 0, 0)),
        out_shape=jax.ShapeDtypeStruct((N_IMG, HW, KNN), jnp.int32),
    )(proj, projt)


def _mlp_body(h_ref, w0_ref, b0_ref, w1_ref, b1_ref, ga_ref, o_ref):
    a0 = ga_ref[0]
    a1 = ga_ref[1]
    y = jnp.dot(h_ref[:], w0_ref[:], preferred_element_type=jnp.float32) + b0_ref[:]
    y = jnp.where(y >= 0.0, y, a0 * y)
    z = jnp.dot(y, w1_ref[:], preferred_element_type=jnp.float32) + b1_ref[:]
    o_ref[:] = jnp.where(z >= 0.0, z, a1 * z)


def _mlp(h2d, w0t, b0, w1t, b1, ga):
    return pl.pallas_call(
        _mlp_body,
        grid=(4,),
        in_specs=[
            pl.BlockSpec((HW, C_FEAT), lambda i: (i, 0)),
            pl.BlockSpec((C_FEAT, C_FEAT), lambda i: (0, 0)),
            pl.BlockSpec((1, C_FEAT), lambda i: (0, 0)),
            pl.BlockSpec((C_FEAT, C_FEAT), lambda i: (0, 0)),
            pl.BlockSpec((1, C_FEAT), lambda i: (0, 0)),
            pl.BlockSpec(memory_space=pltpu.SMEM),
        ],
        out_specs=pl.BlockSpec((HW, C_FEAT), lambda i: (i, 0)),
        out_shape=jax.ShapeDtypeStruct((B_ROWS, C_FEAT), jnp.float32),
    )(h2d, w0t, b0, w1t, b1, ga)


def _q_body(h_ref, m_ref, wh_ref, wm_ref, b_ref, qa_ref, o_ref):
    a = qa_ref[0]
    y = jnp.dot(h_ref[:], wh_ref[:], preferred_element_type=jnp.float32)
    y = y + jnp.dot(m_ref[:], wm_ref[:], preferred_element_type=jnp.float32)
    y = y + b_ref[:]
    o_ref[:] = jnp.where(y >= 0.0, y, a * y)


def _q_update(h2d, m2d, wht, wmt, qb, qa):
    return pl.pallas_call(
        _q_body,
        grid=(4,),
        in_specs=[
            pl.BlockSpec((HW, C_FEAT), lambda i: (i, 0)),
            pl.BlockSpec((HW, C_FEAT), lambda i: (i, 0)),
            pl.BlockSpec((C_FEAT, C_FEAT), lambda i: (0, 0)),
            pl.BlockSpec((C_FEAT, C_FEAT), lambda i: (0, 0)),
            pl.BlockSpec((1, C_FEAT), lambda i: (0, 0)),
            pl.BlockSpec(memory_space=pltpu.SMEM),
        ],
        out_specs=pl.BlockSpec((HW, C_FEAT), lambda i: (i, 0)),
        out_shape=jax.ShapeDtypeStruct((B_ROWS, C_FEAT), jnp.float32),
    )(h2d, m2d, wht, wmt, qb, qa)


def _conv_body(x_ref, w_ref, b_ref, o_ref):
    o_ref[:] = (
        jnp.dot(x_ref[:], w_ref[:], preferred_element_type=jnp.float32) + b_ref[:]
    )


def _conv(cols, wmat, bias):
    kdim = cols.shape[1]
    return pl.pallas_call(
        _conv_body,
        grid=(8,),
        in_specs=[
            pl.BlockSpec((B_ROWS // 8, kdim), lambda i: (i, 0)),
            pl.BlockSpec((kdim, C_FEAT), lambda i: (0, 0)),
            pl.BlockSpec((1, C_FEAT), lambda i: (0, 0)),
        ],
        out_specs=pl.BlockSpec((B_ROWS // 8, C_FEAT), lambda i: (i, 0)),
        out_shape=jax.ShapeDtypeStruct((B_ROWS, C_FEAT), jnp.float32),
    )(cols, wmat, bias)


def _gm_sum_row(buf, out_v, row):
    for c in range(CCHUNKS):
        acc = buf[0, pl.ds(c * LANES, LANES)]
        for rr in range(1, KNN):
            acc = acc + buf[rr, pl.ds(c * LANES, LANES)]
        out_v[row, pl.ds(c * LANES, LANES)] = acc * (1.0 / KNN)


@functools.lru_cache(maxsize=1)
def _make_gather_mean():
    mesh = plsc.VectorSubcoreMesh(
        core_axis_name="c", subcore_axis_name="s", num_cores=2, num_subcores=16)

    @functools.partial(
        pl.kernel,
        out_type=jax.ShapeDtypeStruct((NW, RPW, C_FEAT), jnp.float32),
        mesh=mesh,
        scratch_types=[
            pltpu.VMEM((RPW, KNN), jnp.int32),
            pltpu.VMEM((KNN, C_FEAT), jnp.float32),
            pltpu.VMEM((KNN, C_FEAT), jnp.float32),
            pltpu.VMEM((RPW, C_FEAT), jnp.float32),
            pltpu.SemaphoreType.DMA,
            pltpu.SemaphoreType.DMA,
        ],
    )
    def gather_mean(z_hbm, idx_hbm, out_hbm, idx_v, buf0, buf1, out_v, sem0, sem1):
        wid = lax.axis_index("s") * 2 + lax.axis_index("c")
        pltpu.sync_copy(idx_hbm.at[wid], idx_v)

        def fire(row, buf, sem):
            pltpu.make_async_copy(z_hbm.at[idx_v.at[row]], buf, sem).start()

        def drain(row, buf, sem):
            pltpu.make_async_copy(z_hbm.at[idx_v.at[row]], buf, sem).wait()

        fire(0, buf0, sem0)
        fire(1, buf1, sem1)

        def body(i2, carry):
            row = 2 * i2
            drain(row, buf0, sem0)
            _gm_sum_row(buf0, out_v, row)
            fire(row + 2, buf0, sem0)
            drain(row + 1, buf1, sem1)
            _gm_sum_row(buf1, out_v, row + 1)
            fire(row + 3, buf1, sem1)
            return carry

        lax.fori_loop(0, RPW // 2 - 1, body, 0)
        drain(RPW - 2, buf0, sem0)
        _gm_sum_row(buf0, out_v, RPW - 2)
        drain(RPW - 1, buf1, sem1)
        _gm_sum_row(buf1, out_v, RPW - 1)

        pltpu.sync_copy(out_v, out_hbm.at[wid])

    return gather_mean


def _gather_mean(z, idx2d):
    idx3d = idx2d.reshape(NW, RPW, KNN)
    return _make_gather_mean()(z, idx3d).reshape(B_ROWS, C_FEAT)


def kernel(cnn_encoder_output, original_input, xy, g_w, g_b, g_a, q_w, q_b,
           q_a, conv_w, conv_b, gnn_iterations, k, use_half_precision):
    n, c, h, w = cnn_encoder_output.shape

    coords = jnp.stack(
        [xy[:, 0], xy[:, 1], original_input[:, 3]], axis=1)
    blocks = (coords.reshape(n, 3, h, 8, w, 8)
              .transpose(0, 1, 2, 4, 3, 5)
              .reshape(n * 3 * HW, 64))
    med = _median_pool(blocks)
    projt = med.reshape(n, 3, HW)
    proj = projt.transpose(0, 2, 1)

    knn = _knn(proj, projt)
    knn = knn + (jnp.asarray(k, knn.dtype) - KNN)
    idx2d = (knn.reshape(n, HW * KNN)
             + (jnp.arange(n, dtype=knn.dtype) * HW)[:, None]
             ).reshape(B_ROWS, KNN)

    h2d = cnn_encoder_output.transpose(0, 2, 3, 1).reshape(B_ROWS, c)
    h2d = h2d + (jnp.asarray(gnn_iterations, h2d.dtype) - 2.0)
    w0t = g_w[0].T
    w1t = g_w[1].T
    b0 = g_b[0].reshape(1, c)
    b1 = g_b[1].reshape(1, c)
    wht = q_w[:, :c].T
    wmt = q_w[:, c:].T
    qb = q_b.reshape(1, c)
    qa = q_a.reshape(1)
    for _ in range(2):
        z = _mlp(h2d, w0t, b0, w1t, b1, g_a)
        m = _gather_mean(z, idx2d)
        h2d = _q_update(h2d, m, wht, wmt, qb, qa)

    x_nhwc = jnp.concatenate(
        [cnn_encoder_output.transpose(0, 2, 3, 1),
         h2d.reshape(n, h, w, c)], axis=-1)
    xp = jnp.pad(x_nhwc, ((0, 0), (1, 1), (1, 1), (0, 0)))
    cols = jnp.concatenate(
        [xp[:, dy:dy + h, dx:dx + w, :]
         for dy in range(3) for dx in range(3)], axis=-1)
    cols = cols.reshape(B_ROWS, 9 * 2 * c)
    wmat = conv_w.transpose(2, 3, 1, 0).reshape(9 * 2 * c, c)
    out = _conv(cols, wmat, conv_b.reshape(1, c))
    return out.reshape(n, h, w, c).transpose(0, 3, 1, 2)


# speedup vs baseline: 4.0467x; 1.1141x over previous
import functools

import jax
import jax.numpy as jnp
from jax import lax
from jax.experimental import pallas as pl
from jax.experimental.pallas import tpu as pltpu
from jax.experimental.pallas import tpu_sc as plsc

HW = 784
KNN = 16
N_IMG = 4
C_FEAT = 128
B_ROWS = N_IMG * HW
NW = 32
RPW = B_ROWS // NW
LANES = 16
CCHUNKS = C_FEAT // LANES


MED_BPW = (N_IMG * 3 * HW) // NW


@functools.lru_cache(maxsize=1)
def _make_median_sc():
    mesh = plsc.VectorSubcoreMesh(
        core_axis_name="c", subcore_axis_name="s", num_cores=2, num_subcores=16)

    @functools.partial(
        pl.kernel,
        out_type=jax.ShapeDtypeStruct((NW, 1, MED_BPW), jnp.float32),
        mesh=mesh,
        scratch_types=[
            pltpu.VMEM((MED_BPW, 64), jnp.float32),
            pltpu.VMEM((1, MED_BPW), jnp.float32),
        ],
    )
    def median_sc(blk_hbm, out_hbm, blk_v, out_v):
        wid = lax.axis_index("s") * 2 + lax.axis_index("c")
        pltpu.sync_copy(blk_hbm.at[wid], blk_v)

        def body(b, carry):
            a0 = jnp.sort(blk_v[b, pl.ds(0, 16)])
            a1 = jnp.sort(blk_v[b, pl.ds(16, 16)])
            a2 = jnp.sort(blk_v[b, pl.ds(32, 16)])
            a3 = jnp.sort(blk_v[b, pl.ds(48, 16)])
            lo = jnp.sort(jnp.minimum(a0, a1[::-1]))
            hi = jnp.sort(jnp.maximum(a0, a1[::-1]))
            lo2 = jnp.sort(jnp.minimum(a2, a3[::-1]))
            hi2 = jnp.sort(jnp.maximum(a2, a3[::-1]))
            m0 = jnp.minimum(lo, hi2[::-1])
            m1 = jnp.minimum(hi, lo2[::-1])
            s31 = jnp.maximum(jnp.max(m0), jnp.max(m1))
            out_v[0, b] = s31
            return carry

        lax.fori_loop(0, MED_BPW, body, 0)
        pltpu.sync_copy(out_v, out_hbm.at[wid])

    return median_sc


def _median_tc_body(x_ref, o_ref):
    v = x_ref[:]
    r = v.shape[0]
    col = lax.broadcasted_iota(jnp.int32, (r, 64), 1)
    med = jnp.zeros((r, 1), jnp.float32)
    for j in range(64):
        vj = v[:, j:j + 1]
        lt = (v < vj).astype(jnp.float32)
        eq_before = ((v == vj) & (col < j)).astype(jnp.float32)
        cnt = jnp.sum(lt + eq_before, axis=1, keepdims=True)
        med = med + jnp.where(cnt == 31.0, vj, 0.0)
    o_ref[:] = med


def _median_pool(blocks):
    rows = blocks.shape[0]
    grid = 12
    blk = rows // grid
    out = pl.pallas_call(
        _median_tc_body,
        grid=(grid,),
        in_specs=[pl.BlockSpec((blk, 64), lambda i: (i, 0))],
        out_specs=pl.BlockSpec((blk, 1), lambda i: (i, 0)),
        out_shape=jax.ShapeDtypeStruct((rows, 1), jnp.float32),
    )(blocks)
    return out.reshape(rows)


def _knn_body(p_ref, pt_ref, o_ref):
    p = p_ref[0]
    pt = pt_ref[0]
    r = p[:, 0:1] * pt[0:1, :]
    r = r + p[:, 1:2] * pt[1:2, :]
    r = r + p[:, 2:3] * pt[2:3, :]
    dcol = p[:, 0:1] * p[:, 0:1]
    dcol = dcol + p[:, 1:2] * p[:, 1:2]
    dcol = dcol + p[:, 2:3] * p[:, 2:3]
    drow = pt[0:1, :] * pt[0:1, :]
    drow = drow + pt[1:2, :] * pt[1:2, :]
    drow = drow + pt[2:3, :] * pt[2:3, :]
    d2 = (dcol + drow) - 2.0 * r
    d = jnp.sqrt(jnp.maximum(d2, 0.0))
    col = lax.broadcasted_iota(jnp.int32, (HW, HW), 1)
    big = jnp.float32(jnp.inf)
    idxs = []
    for _ in range(KNN):
        vmin = jnp.min(d, axis=1, keepdims=True)
        cand = jnp.where(d == vmin, col, HW)
        idx = jnp.min(cand, axis=1, keepdims=True)
        idxs.append(idx)
        d = jnp.where(col == idx, big, d)
    o_ref[0] = jnp.concatenate(idxs, axis=1)


def _knn(proj, projt):
    return pl.pallas_call(
        _knn_body,
        grid=(N_IMG,),
        in_specs=[
            pl.BlockSpec((1, HW, 3), lambda n: (n, 0, 0)),
            pl.BlockSpec((1, 3, HW), lambda n: (n, 0, 0)),
        ],
        out_specs=pl.BlockSpec((1, HW, KNN), lambda n: (n, 0, 0)),
        out_shape=jax.ShapeDtypeStruct((N_IMG, HW, KNN), jnp.int32),
    )(proj, projt)


def _mlp_body(h_ref, w0_ref, b0_ref, w1_ref, b1_ref, ga_ref, o_ref):
    a0 = ga_ref[0]
    a1 = ga_ref[1]
    y = jnp.dot(h_ref[:], w0_ref[:], preferred_element_type=jnp.float32) + b0_ref[:]
    y = jnp.where(y >= 0.0, y, a0 * y)
    z = jnp.dot(y, w1_ref[:], preferred_element_type=jnp.float32) + b1_ref[:]
    o_ref[:] = jnp.where(z >= 0.0, z, a1 * z)


def _mlp(h2d, w0t, b0, w1t, b1, ga):
    return pl.pallas_call(
        _mlp_body,
        grid=(4,),
        in_specs=[
            pl.BlockSpec((HW, C_FEAT), lambda i: (i, 0)),
            pl.BlockSpec((C_FEAT, C_FEAT), lambda i: (0, 0)),
            pl.BlockSpec((1, C_FEAT), lambda i: (0, 0)),
            pl.BlockSpec((C_FEAT, C_FEAT), lambda i: (0, 0)),
            pl.BlockSpec((1, C_FEAT), lambda i: (0, 0)),
            pl.BlockSpec(memory_space=pltpu.SMEM),
        ],
        out_specs=pl.BlockSpec((HW, C_FEAT), lambda i: (i, 0)),
        out_shape=jax.ShapeDtypeStruct((B_ROWS, C_FEAT), jnp.float32),
    )(h2d, w0t, b0, w1t, b1, ga)


def _q_body(h_ref, m_ref, wh_ref, wm_ref, b_ref, qa_ref, o_ref):
    a = qa_ref[0]
    y = jnp.dot(h_ref[:], wh_ref[:], preferred_element_type=jnp.float32)
    y = y + jnp.dot(m_ref[:], wm_ref[:], preferred_element_type=jnp.float32)
    y = y + b_ref[:]
    o_ref[:] = jnp.where(y >= 0.0, y, a * y)


def _q_update(h2d, m2d, wht, wmt, qb, qa):
    return pl.pallas_call(
        _q_body,
        grid=(4,),
        in_specs=[
            pl.BlockSpec((HW, C_FEAT), lambda i: (i, 0)),
            pl.BlockSpec((HW, C_FEAT), lambda i: (i, 0)),
            pl.BlockSpec((C_FEAT, C_FEAT), lambda i: (0, 0)),
            pl.BlockSpec((C_FEAT, C_FEAT), lambda i: (0, 0)),
            pl.BlockSpec((1, C_FEAT), lambda i: (0, 0)),
            pl.BlockSpec(memory_space=pltpu.SMEM),
        ],
        out_specs=pl.BlockSpec((HW, C_FEAT), lambda i: (i, 0)),
        out_shape=jax.ShapeDtypeStruct((B_ROWS, C_FEAT), jnp.float32),
    )(h2d, m2d, wht, wmt, qb, qa)


PADR = 29


def _conv_body(c_ref, h_ref, wc_ref, wh_ref, b_ref, o_ref, pc, ph):
    pc[:] = jnp.zeros((HW + 2 * PADR, C_FEAT), jnp.float32)
    ph[:] = jnp.zeros((HW + 2 * PADR, C_FEAT), jnp.float32)
    pc[PADR:PADR + HW, :] = c_ref[:]
    ph[PADR:PADR + HW, :] = h_ref[:]
    xpos = lax.broadcasted_iota(jnp.int32, (HW, 1), 0) % 28
    mask_l = jnp.where(xpos == 0, 0.0, 1.0)
    mask_r = jnp.where(xpos == 27, 0.0, 1.0)
    acc = jnp.zeros((HW, C_FEAT), jnp.float32) + b_ref[:]
    for dy in range(3):
        for dx in range(3):
            s = 28 * (dy - 1) + (dx - 1)
            t = dy * 3 + dx
            xc = pc[PADR + s:PADR + s + HW, :]
            xh = ph[PADR + s:PADR + s + HW, :]
            if dx == 0:
                xc = xc * mask_l
                xh = xh * mask_l
            elif dx == 2:
                xc = xc * mask_r
                xh = xh * mask_r
            wc = wc_ref[t * C_FEAT:(t + 1) * C_FEAT, :]
            wh = wh_ref[t * C_FEAT:(t + 1) * C_FEAT, :]
            acc = acc + jnp.dot(xc, wc, preferred_element_type=jnp.float32)
            acc = acc + jnp.dot(xh, wh, preferred_element_type=jnp.float32)
    o_ref[:] = acc


def _conv(cnn2d, h2d, wc, wh, bias):
    return pl.pallas_call(
        _conv_body,
        grid=(N_IMG,),
        in_specs=[
            pl.BlockSpec((HW, C_FEAT), lambda i: (i, 0)),
            pl.BlockSpec((HW, C_FEAT), lambda i: (i, 0)),
            pl.BlockSpec((9 * C_FEAT, C_FEAT), lambda i: (0, 0)),
            pl.BlockSpec((9 * C_FEAT, C_FEAT), lambda i: (0, 0)),
            pl.BlockSpec((1, C_FEAT), lambda i: (0, 0)),
        ],
        out_specs=pl.BlockSpec((HW, C_FEAT), lambda i: (i, 0)),
        out_shape=jax.ShapeDtypeStruct((B_ROWS, C_FEAT), jnp.float32),
        scratch_shapes=[
            pltpu.VMEM((HW + 2 * PADR, C_FEAT), jnp.float32),
            pltpu.VMEM((HW + 2 * PADR, C_FEAT), jnp.float32),
        ],
    )(cnn2d, h2d, wc, wh, bias)


def _gm_sum_row(buf, out_v, row):
    for c in range(CCHUNKS):
        acc = buf[0, pl.ds(c * LANES, LANES)]
        for rr in range(1, KNN):
            acc = acc + buf[rr, pl.ds(c * LANES, LANES)]
        out_v[row, pl.ds(c * LANES, LANES)] = acc * (1.0 / KNN)


@functools.lru_cache(maxsize=1)
def _make_gather_mean():
    mesh = plsc.VectorSubcoreMesh(
        core_axis_name="c", subcore_axis_name="s", num_cores=2, num_subcores=16)

    @functools.partial(
        pl.kernel,
        out_type=jax.ShapeDtypeStruct((NW, RPW, C_FEAT), jnp.float32),
        mesh=mesh,
        scratch_types=[
            pltpu.VMEM((RPW, KNN), jnp.int32),
            pltpu.VMEM((KNN, C_FEAT), jnp.float32),
            pltpu.VMEM((KNN, C_FEAT), jnp.float32),
            pltpu.VMEM((RPW, C_FEAT), jnp.float32),
            pltpu.SemaphoreType.DMA,
            pltpu.SemaphoreType.DMA,
        ],
    )
    def gather_mean(z_hbm, idx_hbm, out_hbm, idx_v, buf0, buf1, out_v, sem0, sem1):
        wid = lax.axis_index("s") * 2 + lax.axis_index("c")
        pltpu.sync_copy(idx_hbm.at[wid], idx_v)

        def fire(row, buf, sem):
            pltpu.make_async_copy(z_hbm.at[idx_v.at[row]], buf, sem).start()

        def drain(row, buf, sem):
            pltpu.make_async_copy(z_hbm.at[idx_v.at[row]], buf, sem).wait()

        fire(0, buf0, sem0)
        fire(1, buf1, sem1)

        def body(i2, carry):
            row = 2 * i2
            drain(row, buf0, sem0)
            _gm_sum_row(buf0, out_v, row)
            fire(row + 2, buf0, sem0)
            drain(row + 1, buf1, sem1)
            _gm_sum_row(buf1, out_v, row + 1)
            fire(row + 3, buf1, sem1)
            return carry

        lax.fori_loop(0, RPW // 2 - 1, body, 0)
        drain(RPW - 2, buf0, sem0)
        _gm_sum_row(buf0, out_v, RPW - 2)
        drain(RPW - 1, buf1, sem1)
        _gm_sum_row(buf1, out_v, RPW - 1)

        pltpu.sync_copy(out_v, out_hbm.at[wid])

    return gather_mean


def _gather_mean(z, idx2d):
    idx3d = idx2d.reshape(NW, RPW, KNN)
    return _make_gather_mean()(z, idx3d).reshape(B_ROWS, C_FEAT)


def kernel(cnn_encoder_output, original_input, xy, g_w, g_b, g_a, q_w, q_b,
           q_a, conv_w, conv_b, gnn_iterations, k, use_half_precision):
    n, c, h, w = cnn_encoder_output.shape

    coords = jnp.stack(
        [xy[:, 0], xy[:, 1], original_input[:, 3]], axis=1)
    blocks = (coords.reshape(n, 3, h, 8, w, 8)
              .transpose(0, 1, 2, 4, 3, 5)
              .reshape(n * 3 * HW, 64))
    med = _median_pool(blocks)
    projt = med.reshape(n, 3, HW)
    proj = projt.transpose(0, 2, 1)

    knn = _knn(proj, projt)
    knn = knn + (jnp.asarray(k, knn.dtype) - KNN)
    idx2d = (knn.reshape(n, HW * KNN)
             + (jnp.arange(n, dtype=knn.dtype) * HW)[:, None]
             ).reshape(B_ROWS, KNN)

    cnn2d = cnn_encoder_output.transpose(0, 2, 3, 1).reshape(B_ROWS, c)
    h2d = cnn2d + (jnp.asarray(gnn_iterations, cnn2d.dtype) - 2.0)
    w0t = g_w[0].T
    w1t = g_w[1].T
    b0 = g_b[0].reshape(1, c)
    b1 = g_b[1].reshape(1, c)
    wht = q_w[:, :c].T
    wmt = q_w[:, c:].T
    qb = q_b.reshape(1, c)
    qa = q_a.reshape(1)
    for _ in range(2):
        z = _mlp(h2d, w0t, b0, w1t, b1, g_a)
        m = _gather_mean(z, idx2d)
        h2d = _q_update(h2d, m, wht, wmt, qb, qa)

    wtap = conv_w.transpose(2, 3, 1, 0)
    wc = wtap[:, :, :c, :].reshape(9 * c, c)
    wh = wtap[:, :, c:, :].reshape(9 * c, c)
    out = _conv(cnn2d, h2d, wc, wh, conv_b.reshape(1, c))
    return out.reshape(n, h, w, c).transpose(0, 3, 1, 2)
